# baseline (speedup 1.0000x reference)
import functools

import jax
import jax.numpy as jnp
from jax import lax
from jax.experimental import pallas as pl
from jax.experimental.pallas import tpu as pltpu
from jax.experimental.pallas import tpu_sc as plsc

_B, _S, _V, _D = 16, 4096, 272, 512
_NW = 32
_HALF = _S // 2
_C = 64
_NCHUNK = _HALF // _C
_NPAIR = _NCHUNK // 2
_L = 16

_CODE_START = 256
_CODE_END = 257
_THINK_START = 259
_THINK_END = 260
_ADDR_KEY = 206
_MARK_TS = 456
_MARK_TE = 457


def _sc_body(tok_hbm, w_hbm, out_hbm, w_sh, tokrow,
             rows0, rows1, gsem0, gsem1, osem0, osem1):
    cid = lax.axis_index("c")
    sid = lax.axis_index("s")
    wid = sid * 2 + cid
    b = wid // 2
    h = wid % 2
    row_base = b * _S

    @pl.when(sid == 0)
    def _():
        pltpu.sync_copy(w_hbm, w_sh)

    pltpu.sync_copy(tok_hbm.at[pl.ds(row_base, _S)], tokrow)

    iota = lax.iota(jnp.int32, _L)
    neg1 = jnp.asarray(-1, jnp.int32)
    ones = jnp.full((_L,), 1.0, jnp.float32)

    def pref_body(j, carry):
        ccs, cce = carry
        t = tokrow[pl.ds(j * _L, _L)]
        pos = iota + j * _L
        ccs = jnp.maximum(ccs, jnp.max(jnp.where(t == _CODE_START, pos, -1)))
        cce = jnp.maximum(cce, jnp.max(jnp.where(t == _CODE_END, pos, -1)))
        return ccs, cce

    ccs0, cce0 = lax.fori_loop(0, h * (_HALF // _L), pref_body, (neg1, neg1))

    plsc.subcore_barrier()

    def start_gathers(c, rows, gsem):
        off = h * _HALF + c * _C
        for j in range(_C // _L):
            tvec = tokrow[pl.ds(off + j * _L, _L)]
            for i in range(_L):
                pltpu.async_copy(w_sh.at[tvec[i]], rows.at[j * _L + i], gsem)

    def drain_gathers(rows, gsem):
        pltpu.make_async_copy(w_sh.at[pl.ds(0, _C)], rows, gsem).wait()

    def modify(c, rows, carry):
        ccs, cce = carry
        off = h * _HALF + c * _C
        for j in range(_C // _L):
            t = tokrow[pl.ds(off + j * _L, _L)]
            pos = iota + (off + j * _L)
            last_cs = jnp.maximum(
                plsc.cummax(jnp.where(t == _CODE_START, pos, neg1)), ccs)
            ccs = jnp.max(last_cs)
            last_ce = jnp.maximum(
                plsc.cummax(jnp.where(t == _CODE_END, pos, neg1)), cce)
            cce = jnp.max(last_ce)
            seq_pos = pos - last_cs - 1
            byte_off = seq_pos & 7
            mask = ((last_cs >= 0) & (last_ce < 0) & (t < 256)
                    & (seq_pos >= 0) & (byte_off < 5))
            addr = jnp.where(mask, (seq_pos >> 3) * 8 + 2 + byte_off, 0)
            slot = iota + j * _L
            plsc.store_scatter(rows, [slot, _ADDR_KEY + (addr & 15)],
                               ones, mask=mask)
            plsc.store_scatter(rows, [slot, _ADDR_KEY + 16 + ((addr >> 4) & 15)],
                               ones, mask=mask)
            plsc.store_scatter(rows, [slot, _ADDR_KEY + 32 + ((addr >> 8) & 15)],
                               ones, mask=mask)
            ts = t == _THINK_START
            te = t == _THINK_END
            plsc.store_scatter(rows, [slot, jnp.where(ts, _MARK_TS, _MARK_TE)],
                               ones, mask=ts | te)
        return ccs, cce

    bufs = ((rows0, gsem0, osem0), (rows1, gsem1, osem1))
    out_half = row_base + h * _HALF

    def out_at(c):
        return out_hbm.at[pl.ds(out_half + c * _C, _C)]

    start_gathers(0, rows0, gsem0)

    def pair_body(p, carry):
        for k in range(2):
            c = p * 2 + k
            rowsb, gsem, osem = bufs[k]
            rowsn, gsemn, osemn = bufs[1 - k]
            drain_gathers(rowsb, gsem)
            carry = modify(c, rowsb, carry)
            pltpu.async_copy(rowsb, out_at(c), osem)
            if k == 0:
                @pl.when(p > 0)
                def _():
                    pltpu.make_async_copy(rowsn, out_at(c - 1), osemn).wait()

                start_gathers(c + 1, rowsn, gsemn)
            else:
                pltpu.make_async_copy(rowsn, out_at(c - 1), osemn).wait()

                @pl.when(p < _NPAIR - 1)
                def _():
                    start_gathers(c + 1, rowsn, gsemn)
        return carry

    lax.fori_loop(0, _NPAIR, pair_body, (ccs0, cce0))
    pltpu.make_async_copy(rows1, out_at(_NCHUNK - 1), osem1).wait()


def kernel(token_ids, W):
    mesh = plsc.VectorSubcoreMesh(core_axis_name="c", subcore_axis_name="s")
    run = functools.partial(
        pl.kernel,
        out_type=jax.ShapeDtypeStruct((_B * _S, _D), jnp.float32),
        mesh=mesh,
        compiler_params=pltpu.CompilerParams(
            needs_layout_passes=False,
            disable_bounds_checks=True,
            disable_semaphore_checks=True,
            skip_device_barrier=True,
        ),
        scratch_types=[
            pltpu.VMEM_SHARED((_V, _D), jnp.float32),
            pltpu.VMEM((_S,), jnp.int32),
            pltpu.VMEM((_C, _D), jnp.float32),
            pltpu.VMEM((_C, _D), jnp.float32),
            pltpu.SemaphoreType.DMA,
            pltpu.SemaphoreType.DMA,
            pltpu.SemaphoreType.DMA,
            pltpu.SemaphoreType.DMA,
        ],
    )(_sc_body)
    out = run(token_ids.reshape(-1), W)
    return out.reshape(_B, _S, _D)


# speedup vs baseline: 1.0244x; 1.0244x over previous
import functools

import jax
import jax.numpy as jnp
from jax import lax
from jax.experimental import pallas as pl
from jax.experimental.pallas import tpu as pltpu
from jax.experimental.pallas import tpu_sc as plsc

_B, _S, _V, _D = 16, 4096, 272, 512
_NW = 32
_HALF = _S // 2
_C = 64
_NCHUNK = _HALF // _C
_NPAIR = _NCHUNK // 2
_L = 16

_CODE_START = 256
_CODE_END = 257
_THINK_START = 259
_THINK_END = 260
_ADDR_KEY = 206
_MARK_TS = 456
_MARK_TE = 457


def _sc_body(tok_hbm, w_hbm, out_hbm, w_sh, tokrow,
             rows0, rows1, gsem0, gsem1, osem0, osem1):
    cid = lax.axis_index("c")
    sid = lax.axis_index("s")
    wid = sid * 2 + cid
    b = wid // 2
    h = wid % 2
    row_base = b * _S

    @pl.when(sid == 0)
    def _():
        pltpu.sync_copy(w_hbm, w_sh)

    pltpu.sync_copy(tok_hbm.at[pl.ds(row_base, _S)], tokrow)

    iota = lax.iota(jnp.int32, _L)
    neg1 = jnp.asarray(-1, jnp.int32)
    ones = jnp.full((_L,), 1.0, jnp.float32)

    def pref_body(j, carry):
        ccs, cce = carry
        t = tokrow[pl.ds(j * _L, _L)]
        pos = iota + j * _L
        ccs = jnp.maximum(ccs, jnp.max(jnp.where(t == _CODE_START, pos, -1)))
        cce = jnp.maximum(cce, jnp.max(jnp.where(t == _CODE_END, pos, -1)))
        return ccs, cce

    ccs0, cce0 = lax.fori_loop(0, h * (_HALF // _L), pref_body, (neg1, neg1))

    plsc.subcore_barrier()

    def start_gathers(c, rows, gsem):
        off = h * _HALF + c * _C
        for j in range(_C // _L):
            tvec = tokrow[pl.ds(off + j * _L, _L)]
            for i in range(_L):
                pltpu.async_copy(w_sh.at[tvec[i]], rows.at[j * _L + i], gsem)

    def drain_gathers(rows, gsem):
        pltpu.make_async_copy(w_sh.at[pl.ds(0, _C)], rows, gsem).wait()

    def modify(c, rows, carry):
        ccs, cce = carry
        off = h * _HALF + c * _C
        for j in range(_C // _L):
            t = tokrow[pl.ds(off + j * _L, _L)]
            pos = iota + (off + j * _L)
            last_cs = jnp.maximum(
                plsc.cummax(jnp.where(t == _CODE_START, pos, neg1)), ccs)
            ccs = jnp.max(last_cs)
            last_ce = jnp.maximum(
                plsc.cummax(jnp.where(t == _CODE_END, pos, neg1)), cce)
            cce = jnp.max(last_ce)
            seq_pos = pos - last_cs - 1
            byte_off = seq_pos & 7
            mask = ((last_cs >= 0) & (last_ce < 0) & (t < 256)
                    & (seq_pos >= 0) & (byte_off < 5))
            addr = jnp.where(mask, (seq_pos >> 3) * 8 + 2 + byte_off, 0)
            slot = iota + j * _L
            plsc.store_scatter(rows, [slot, _ADDR_KEY + (addr & 15)],
                               ones, mask=mask)
            plsc.store_scatter(rows, [slot, _ADDR_KEY + 16 + ((addr >> 4) & 15)],
                               ones, mask=mask)
            plsc.store_scatter(rows, [slot, _ADDR_KEY + 32 + ((addr >> 8) & 15)],
                               ones, mask=mask)
            ts = t == _THINK_START
            te = t == _THINK_END
            plsc.store_scatter(rows, [slot, jnp.where(ts, _MARK_TS, _MARK_TE)],
                               ones, mask=ts | te)
        return ccs, cce

    bufs = ((rows0, gsem0, osem0), (rows1, gsem1, osem1))
    out_half = row_base + h * _HALF

    def out_at(c):
        return out_hbm.at[pl.ds(out_half + c * _C, _C)]

    start_gathers(0, rows0, gsem0)

    def pair_body(p, carry):
        for k in range(2):
            c = p * 2 + k
            rowsb, gsem, osem = bufs[k]
            rowsn, gsemn, osemn = bufs[1 - k]
            drain_gathers(rowsb, gsem)
            carry = modify(c, rowsb, carry)
            pltpu.async_copy(rowsb, out_at(c), osem)
            if k == 0:
                @pl.when(p > 0)
                def _():
                    pltpu.make_async_copy(rowsn, out_at(c - 1), osemn).wait()

                start_gathers(c + 1, rowsn, gsemn)
            else:
                pltpu.make_async_copy(rowsn, out_at(c - 1), osemn).wait()

                @pl.when(p < _NPAIR - 1)
                def _():
                    start_gathers(c + 1, rowsn, gsemn)
        return carry

    lax.fori_loop(0, _NPAIR, pair_body, (ccs0, cce0))
    pltpu.make_async_copy(rows1, out_at(_NCHUNK - 1), osem1).wait()


def kernel(token_ids, W):
    mesh = plsc.VectorSubcoreMesh(core_axis_name="c", subcore_axis_name="s")
    run = functools.partial(
        pl.kernel,
        out_type=jax.ShapeDtypeStruct((_B * _S, _D), jnp.float32),
        mesh=mesh,
        compiler_params=pltpu.CompilerParams(needs_layout_passes=False),
        scratch_types=[
            pltpu.VMEM_SHARED((_V, _D), jnp.float32),
            pltpu.VMEM((_S,), jnp.int32),
            pltpu.VMEM((_C, _D), jnp.float32),
            pltpu.VMEM((_C, _D), jnp.float32),
            pltpu.SemaphoreType.DMA,
            pltpu.SemaphoreType.DMA,
            pltpu.SemaphoreType.DMA,
            pltpu.SemaphoreType.DMA,
        ],
    )(_sc_body)
    out = run(token_ids.reshape(-1), W)
    return out.reshape(_B, _S, _D)
